# baseline (speedup 1.0000x reference)
import functools

import jax
import jax.numpy as jnp
from jax import lax
from jax.experimental import pallas as pl
from jax.experimental.pallas import tpu as pltpu
from jax.experimental.pallas import tpu_sc as plsc

NM = 1_000_000
EMBED = 32
BATCH = 16384
ROWS4 = NM // 4
NC, NS, L = 2, 16, 16
NW = NC * NS
PAIRS_W = BATCH // NW
SLOTS_W = 2 * PAIRS_W
HALF = PAIRS_W // 2
CHUNK = 128
HGROUPS = HALF // L

_mesh = plsc.VectorSubcoreMesh(core_axis_name="c", subcore_axis_name="s")


@functools.partial(
    pl.kernel,
    out_type=jax.ShapeDtypeStruct((BATCH,), jnp.float32),
    mesh=_mesh,
    scratch_types=[
        pltpu.VMEM((SLOTS_W,), jnp.int32),
        pltpu.VMEM((SLOTS_W,), jnp.int32),
        pltpu.VMEM((SLOTS_W,), jnp.int32),
        pltpu.VMEM((2 * HALF, 128), jnp.float32),
        pltpu.VMEM((SLOTS_W,), jnp.float32),
        pltpu.VMEM((PAIRS_W,), jnp.float32),
        pltpu.SemaphoreType.DMA,
        pltpu.SemaphoreType.DMA,
    ],
    compiler_params=pltpu.CompilerParams(
        needs_layout_passes=False, use_tc_tiling_on_sc=True),
)
def _cf_kernel(u_hbm, v_hbm, bias_hbm, ent_hbm, out_hbm,
               uv_idx, rowidx, colsel, rows, bvals, out_v, sem_e, sem_b):
    wid = lax.axis_index("s") * NC + lax.axis_index("c")
    base = wid * PAIRS_W
    pltpu.sync_copy(u_hbm.at[pl.ds(base, PAIRS_W)],
                    uv_idx.at[pl.ds(0, PAIRS_W)])
    pltpu.sync_copy(v_hbm.at[pl.ds(base, PAIRS_W)],
                    uv_idx.at[pl.ds(PAIRS_W, PAIRS_W)])

    bias_copies = []
    for c in range(SLOTS_W // CHUNK):
        sl = pl.ds(c * CHUNK, CHUNK)
        bias_copies.append(
            pltpu.async_copy(bias_hbm.at[uv_idx.at[sl]], bvals.at[sl], sem_b))

    def prep_body(k, carry):
        sl = pl.ds(k * L, L)
        uv = uv_idx[sl]
        rowidx[sl] = lax.shift_right_logical(uv, 2)
        colsel[sl] = lax.shift_left(lax.bitwise_and(uv, 3), 5)
        return carry

    lax.fori_loop(0, SLOTS_W // L, prep_body, 0)

    lanes = lax.iota(jnp.int32, L)

    def half_body(h, carry):
        for part in range(2):
            for c in range(HALF // CHUNK):
                src = pl.ds(part * PAIRS_W + h * HALF + c * CHUNK, CHUNK)
                dst = pl.ds(part * HALF + c * CHUNK, CHUNK)
                pltpu.async_copy(
                    ent_hbm.at[rowidx.at[src]], rows.at[dst], sem_e)
        for _ in range(2 * (HALF // CHUNK)):
            pltpu.make_async_copy(
                ent_hbm.at[rowidx.at[pl.ds(0, CHUNK)]],
                rows.at[pl.ds(0, CHUNK)], sem_e).wait()

        def group_body(g, carry2):
            p = h * HALF + g * L
            slotu = g * L + lanes
            slotv = HALF + g * L + lanes
            cu = colsel[pl.ds(p, L)]
            cv = colsel[pl.ds(PAIRS_W + p, L)]
            acc = bvals[pl.ds(p, L)] + bvals[pl.ds(PAIRS_W + p, L)]
            for t in range(EMBED):
                ut = plsc.load_gather(rows, [slotu, cu + t])
                vt = plsc.load_gather(rows, [slotv, cv + t])
                acc = acc + ut * vt
            out_v[pl.ds(p, L)] = acc
            return carry2

        lax.fori_loop(0, HGROUPS, group_body, 0)
        return carry

    for cp in bias_copies:
        cp.wait()
    lax.fori_loop(0, 2, half_body, 0)
    pltpu.sync_copy(out_v, out_hbm.at[pl.ds(base, PAIRS_W)])


def kernel(x, biases, entities):
    x = x.astype(jnp.int32)
    e4 = entities.reshape(ROWS4, 128)
    return _cf_kernel(x[:, 0], x[:, 1], biases.reshape(-1), e4)


# speedup vs baseline: 1.0245x; 1.0245x over previous
import functools

import jax
import jax.numpy as jnp
from jax import lax
from jax.experimental import pallas as pl
from jax.experimental.pallas import tpu as pltpu
from jax.experimental.pallas import tpu_sc as plsc

NM = 1_000_000
EMBED = 32
BATCH = 16384
NC, NS, L = 2, 16, 16
NW = NC * NS
PAIRS_W = BATCH // NW
SLOTS_W = 2 * PAIRS_W
HALF = PAIRS_W // 2
CHUNK = 128
HGROUPS = HALF // L

_mesh = plsc.VectorSubcoreMesh(core_axis_name="c", subcore_axis_name="s")


@functools.partial(
    pl.kernel,
    out_type=jax.ShapeDtypeStruct((BATCH,), jnp.float32),
    mesh=_mesh,
    scratch_types=[
        pltpu.VMEM((SLOTS_W,), jnp.int32),
        pltpu.VMEM((2 * HALF, 128), jnp.float32),
        pltpu.VMEM((SLOTS_W,), jnp.float32),
        pltpu.VMEM((PAIRS_W,), jnp.float32),
        pltpu.SemaphoreType.DMA,
        pltpu.SemaphoreType.DMA,
    ],
    compiler_params=pltpu.CompilerParams(
        needs_layout_passes=False, use_tc_tiling_on_sc=True),
)
def _cf_kernel(u_hbm, v_hbm, bias_hbm, ent_hbm, out_hbm,
               uv_idx, rows, bvals, out_v, sem_e, sem_b):
    wid = lax.axis_index("s") * NC + lax.axis_index("c")
    base = wid * PAIRS_W
    pltpu.sync_copy(u_hbm.at[pl.ds(base, PAIRS_W)],
                    uv_idx.at[pl.ds(0, PAIRS_W)])
    pltpu.sync_copy(v_hbm.at[pl.ds(base, PAIRS_W)],
                    uv_idx.at[pl.ds(PAIRS_W, PAIRS_W)])

    bias_copies = []
    for c in range(SLOTS_W // CHUNK):
        sl = pl.ds(c * CHUNK, CHUNK)
        bias_copies.append(
            pltpu.async_copy(bias_hbm.at[uv_idx.at[sl]], bvals.at[sl], sem_b))

    lanes = lax.iota(jnp.int32, L)

    def half_body(h, carry):
        for part in range(2):
            for c in range(HALF // CHUNK):
                src = pl.ds(part * PAIRS_W + h * HALF + c * CHUNK, CHUNK)
                dst = pl.ds(part * HALF + c * CHUNK, CHUNK)
                pltpu.async_copy(
                    ent_hbm.at[uv_idx.at[src]], rows.at[dst], sem_e)
        for _ in range(2 * (HALF // CHUNK)):
            pltpu.make_async_copy(
                ent_hbm.at[uv_idx.at[pl.ds(0, CHUNK)]],
                rows.at[pl.ds(0, CHUNK)], sem_e).wait()

        def group_body(g, carry2):
            p = h * HALF + g * L
            slotu = g * L + lanes
            slotv = HALF + g * L + lanes
            acc = bvals[pl.ds(p, L)] + bvals[pl.ds(PAIRS_W + p, L)]
            for t in range(EMBED):
                col = jnp.full((L,), t, jnp.int32)
                ut = plsc.load_gather(rows, [slotu, col])
                vt = plsc.load_gather(rows, [slotv, col])
                acc = acc + ut * vt
            out_v[pl.ds(p, L)] = acc
            return carry2

        lax.fori_loop(0, HGROUPS, group_body, 0)
        return carry

    for cp in bias_copies:
        cp.wait()
    lax.fori_loop(0, 2, half_body, 0)
    pltpu.sync_copy(out_v, out_hbm.at[pl.ds(base, PAIRS_W)])


def kernel(x, biases, entities):
    x = x.astype(jnp.int32)
    e_pad = jnp.pad(entities, ((0, 0), (0, 128 - EMBED)))
    return _cf_kernel(x[:, 0], x[:, 1], biases.reshape(-1), e_pad)
